# baseline (speedup 1.0000x reference)
import functools

import jax
import jax.numpy as jnp
from jax import lax
from jax.experimental import pallas as pl
from jax.experimental.pallas import tpu as pltpu
from jax.experimental.pallas import tpu_sc as plsc

B, L, H = 8, 512, 16
V = 513
SUPER = 512
W = 520
NW = 32
Q = 4
ROWS_PER_W = L // Q
CH_ROWS = 4
N_CHUNKS = ROWS_PER_W // CH_ROWS


def _sc_body(pos_hbm, tab_hbm, out_hbm, tab_v, idx_v, out_v, sem_in, sem_out):
    wid = lax.axis_index("s") * 2 + lax.axis_index("c")
    b = wid // Q
    row0 = (wid % Q) * ROWS_PER_W

    pltpu.sync_copy(tab_hbm, tab_v)
    lane = lax.iota(jnp.int32, 16)
    sup = jnp.full((16,), SUPER, jnp.int32)

    def idx_copy(chunk, buf):
        r = row0 + chunk * CH_ROWS
        return pltpu.make_async_copy(
            pos_hbm.at[b, pl.ds(r, CH_ROWS), :], idx_v.at[buf], sem_in
        )

    def out_copy(chunk, buf):
        r = row0 + chunk * CH_ROWS
        return pltpu.make_async_copy(
            out_v.at[buf], out_hbm.at[b, :, pl.ds(r, CH_ROWS), :], sem_out
        )

    idx_copy(0, 0).start()
    idx_copy(1, 1).start()

    def process(t, chunk, buf):
        idx_copy(chunk, buf).wait()

        @pl.when(t > 0)
        def _drain():
            out_copy(chunk, buf).wait()

        r_glob = row0 + chunk * CH_ROWS
        for r in range(CH_ROWS):
            head16 = idx_v[buf, r, pl.ds(0, 16)]
            idx_v[buf, r, pl.ds(0, 16)] = jnp.where(lane == 0, sup, head16)

        @pl.when(r_glob == 0)
        def _fix_row0():
            @plsc.parallel_loop(0, L, 16, unroll=4)
            def fill(o):
                o = pl.multiple_of(o, 16)
                idx_v[buf, 0, pl.ds(o, 16)] = sup

        for r in range(CH_ROWS):
            @plsc.parallel_loop(0, L, 16, unroll=8)
            def vec_body(o):
                o = pl.multiple_of(o, 16)
                idxf = idx_v[buf, r, pl.ds(o, 16)]
                for h in range(H):
                    vals = plsc.load_gather(tab_v, [idxf + (h * W)])
                    out_v[buf, h, r, pl.ds(o, 16)] = vals

        out_copy(chunk, buf).start()

        @pl.when(t < N_CHUNKS // 2 - 1)
        def _prefetch():
            idx_copy(chunk + 2, buf).start()

    def pair_body(t, c):
        process(t, 2 * t, 0)
        process(t, 2 * t + 1, 1)
        return c

    lax.fori_loop(0, N_CHUNKS // 2, pair_body, 0)
    out_copy(0, 0).wait()
    out_copy(0, 1).wait()


def kernel(spatial_pos, spatial_embeddings):
    tab = (
        jnp.zeros((H, W), jnp.float32)
        .at[:, :V].set(spatial_embeddings.T)
        .reshape(H * W)
    )
    mesh = plsc.VectorSubcoreMesh(
        core_axis_name="c", subcore_axis_name="s", num_cores=2, num_subcores=16
    )
    run = functools.partial(
        pl.kernel,
        out_type=jax.ShapeDtypeStruct((B, H, L, L), jnp.float32),
        mesh=mesh,
        scratch_types=[
            pltpu.VMEM((H * W,), jnp.float32),
            pltpu.VMEM((2, CH_ROWS, L), jnp.int32),
            pltpu.VMEM((2, H, CH_ROWS, L), jnp.float32),
            pltpu.SemaphoreType.DMA,
            pltpu.SemaphoreType.DMA,
        ],
        compiler_params=pltpu.CompilerParams(needs_layout_passes=False),
    )(_sc_body)
    return run(spatial_pos, tab)


# speedup vs baseline: 1.2239x; 1.2239x over previous
import functools

import jax
import jax.numpy as jnp
from jax import lax
from jax.experimental import pallas as pl
from jax.experimental.pallas import tpu as pltpu
from jax.experimental.pallas import tpu_sc as plsc

B, L, H = 8, 512, 16
V = 513
SUPER = 512
W = 520
NW = 32
Q = 4
ROWS_PER_W = L // Q
CH_ROWS = 4
N_CHUNKS = ROWS_PER_W // CH_ROWS


def _sc_body(pos_hbm, tab_hbm, out_hbm, tab_v, idx_v, out_v, sem_in, sem_out):
    wid = lax.axis_index("s") * 2 + lax.axis_index("c")
    b = wid // Q
    row0 = (wid % Q) * ROWS_PER_W

    pltpu.sync_copy(tab_hbm, tab_v)
    lane = lax.iota(jnp.int32, 16)
    sup = jnp.full((16,), SUPER, jnp.int32)

    def idx_copy(chunk, buf):
        r = row0 + chunk * CH_ROWS
        return pltpu.make_async_copy(
            pos_hbm.at[b, pl.ds(r, CH_ROWS), :], idx_v.at[buf], sem_in
        )

    def out_copy(chunk, buf):
        r = row0 + chunk * CH_ROWS
        return pltpu.make_async_copy(
            out_v.at[buf], out_hbm.at[b, :, pl.ds(r, CH_ROWS), :], sem_out
        )

    idx_copy(0, 0).start()
    idx_copy(1, 1).start()

    def process(t, chunk, buf):
        idx_copy(chunk, buf).wait()

        @pl.when(t > 0)
        def _drain():
            out_copy(chunk, buf).wait()

        r_glob = row0 + chunk * CH_ROWS
        for r in range(CH_ROWS):
            head16 = idx_v[buf, r, pl.ds(0, 16)]
            idx_v[buf, r, pl.ds(0, 16)] = jnp.where(lane == 0, sup, head16)

        @pl.when(r_glob == 0)
        def _fix_row0():
            @plsc.parallel_loop(0, L, 16, unroll=4)
            def fill(o):
                o = pl.multiple_of(o, 16)
                idx_v[buf, 0, pl.ds(o, 16)] = sup

        for r in range(CH_ROWS):
            @plsc.parallel_loop(0, L, 16, unroll=4)
            def vec_body(o):
                o = pl.multiple_of(o, 16)
                idxf = idx_v[buf, r, pl.ds(o, 16)]
                for h in range(H):
                    vals = plsc.load_gather(tab_v.at[pl.ds(h * W, W)], [idxf])
                    out_v[buf, h, r, pl.ds(o, 16)] = vals

        out_copy(chunk, buf).start()

        @pl.when(t < N_CHUNKS // 2 - 1)
        def _prefetch():
            idx_copy(chunk + 2, buf).start()

    def pair_body(t, c):
        process(t, 2 * t, 0)
        process(t, 2 * t + 1, 1)
        return c

    lax.fori_loop(0, N_CHUNKS // 2, pair_body, 0)
    out_copy(0, 0).wait()
    out_copy(0, 1).wait()


def kernel(spatial_pos, spatial_embeddings):
    tab = (
        jnp.zeros((H, W), jnp.float32)
        .at[:, :V].set(spatial_embeddings.T)
        .reshape(H * W)
    )
    mesh = plsc.VectorSubcoreMesh(
        core_axis_name="c", subcore_axis_name="s", num_cores=2, num_subcores=16
    )
    run = functools.partial(
        pl.kernel,
        out_type=jax.ShapeDtypeStruct((B, H, L, L), jnp.float32),
        mesh=mesh,
        scratch_types=[
            pltpu.VMEM((H * W,), jnp.float32),
            pltpu.VMEM((2, CH_ROWS, L), jnp.int32),
            pltpu.VMEM((2, H, CH_ROWS, L), jnp.float32),
            pltpu.SemaphoreType.DMA,
            pltpu.SemaphoreType.DMA,
        ],
        compiler_params=pltpu.CompilerParams(needs_layout_passes=False),
    )(_sc_body)
    return run(spatial_pos, tab)
